# baseline (speedup 1.0000x reference)
import functools
import math

import jax
import jax.numpy as jnp
from jax import lax
from jax.experimental import pallas as pl
from jax.experimental.pallas import tpu as pltpu
from jax.experimental.pallas import tpu_sc as plsc

POI_NUM = 1000000
CATE_NUM = 1000
EMBED_DIM = 64
FREQ_DIM = 16
BATCH = 16384
SEQ = 50
N = BATCH * SEQ
NW = 32
EPT = N // NW
CHE = 400
CHR = CHE // SEQ
NCH = EPT // CHE
NG = CHE // 16

SCALE = 500.0 * math.pi
LON_MIN, LON_DEN = -180.0, 360.0
LAT_MIN, LAT_DEN = -90.0, 180.0
INV2PI = 1.0 / (2.0 * math.pi)
TWOPI = 2.0 * math.pi

S3, S5, S7 = (-0.16620338965808842, 0.00806673837856976,
              -0.0001515486170838437)
C2, C4, C6, C8 = (-0.4998795971710147, 0.04153852455139196,
                  -0.0013464001942717439, 1.9172344065239433e-05)


def _f32(x):
  return jnp.float32(x)


def _sincos(r):
  x2 = r * r
  s = _f32(S7)
  s = s * x2 + _f32(S5)
  s = s * x2 + _f32(S3)
  s = s * x2 + _f32(1.0)
  s = s * r
  c = _f32(C8)
  c = c * x2 + _f32(C6)
  c = c * x2 + _f32(C4)
  c = c * x2 + _f32(C2)
  c = c * x2 + _f32(1.0)
  return s, c


def _body(poi_h, cid_h, lon_h, lat_h, idt_h, ct_h, out_h,
          ctv, idxv0, idxv1, cidv0, cidv1, lonv0, lonv1, latv0, latv1,
          rows0, rows1, sin0, sin1, sg0, sg1, so0, so1):
  wid = lax.axis_index("s") * 2 + lax.axis_index("c")
  ebase = wid * EPT
  rbase = wid * (EPT // SEQ)
  idxv = (idxv0, idxv1)
  cidv = (cidv0, cidv1)
  lonv = (lonv0, lonv1)
  latv = (latv0, latv1)
  rows = (rows0, rows1)
  sin_ = (sin0, sin1)
  sg = (sg0, sg1)
  so = (so0, so1)

  pltpu.sync_copy(ct_h, ctv)

  iota = lax.iota(jnp.int32, 16)
  freqs = jnp.exp(iota.astype(jnp.float32) * _f32(-(math.log(10000.0) / FREQ_DIM)))
  lon_f1 = freqs * _f32(SCALE / LON_DEN)
  lon_g1 = freqs * _f32(-LON_MIN / LON_DEN * SCALE)
  lat_f1 = freqs * _f32(SCALE / LAT_DEN)
  lat_g1 = freqs * _f32(-LAT_MIN / LAT_DEN * SCALE)
  lon_f2 = lon_f1 * _f32(INV2PI)
  lon_g2 = lon_g1 * _f32(INV2PI) + _f32(0.5)
  lat_f2 = lat_f1 * _f32(INV2PI)
  lat_g2 = lat_g1 * _f32(INV2PI) + _f32(0.5)

  GSPLIT = ((0, 128), (128, 128), (256, 128), (384, 16))

  def issue_inputs(c, b):
    e0 = ebase + c * CHE
    return [pltpu.async_copy(src.at[pl.ds(e0, CHE)], dst, sin_[b])
            for src, dst in ((poi_h, idxv[b]), (cid_h, cidv[b]),
                             (lon_h, lonv[b]), (lat_h, latv[b]))]

  def issue_gathers(b):
    for o, n in GSPLIT:
      pltpu.async_copy(idt_h.at[idxv[b].at[pl.ds(o, n)]],
                       rows[b].at[pl.ds(o, n)], sg[b])

  def wait_gathers(b):
    for o, n in GSPLIT:
      pltpu.make_async_copy(idt_h.at[idxv[b].at[pl.ds(o, n)]],
                            rows[b].at[pl.ds(o, n)], sg[b]).wait()

  def issue_outs(c, b):
    r0 = rbase + c * CHR
    for j in range(CHR):
      pltpu.async_copy(rows[b].at[pl.ds(j * SEQ, SEQ)], out_h.at[r0 + j],
                       so[b])

  def wait_outs(b):
    for j in range(CHR):
      pltpu.make_async_copy(rows[b].at[pl.ds(j * SEQ, SEQ)],
                            out_h.at[rbase + j], so[b]).wait()

  def trig(v, f1, g1, f2, g2):
    a = v * f1 + g1
    t = v * f2 + g2
    nf = t.astype(jnp.int32).astype(jnp.float32)
    return _sincos(a - nf * _f32(TWOPI))

  def compute(b):
    @plsc.parallel_loop(0, NG, unroll=5)
    def grp(t, b=b):
      i0 = t * 16
      cidvec = cidv[b][pl.ds(i0, 16)]
      lonvec = lonv[b][pl.ds(i0, 16)]
      latvec = latv[b][pl.ds(i0, 16)]
      for e in range(16):
        lv = jnp.full((16,), lonvec[e], jnp.float32)
        tv = jnp.full((16,), latvec[e], jnp.float32)
        sl, cl = trig(lv, lon_f1, lon_g1, lon_f2, lon_g2)
        st, ct = trig(tv, lat_f1, lat_g1, lat_f2, lat_g2)
        cb = cidvec[e] * jnp.int32(EMBED_DIM)
        i = i0 + e
        for k, tg in enumerate((sl, cl, st, ct)):
          cate_k = ctv[pl.ds(cb + k * 16, 16)]
          cur = rows[b][i, pl.ds(k * 16, 16)]
          rows[b][i, pl.ds(k * 16, 16)] = cur + cate_k + tg

  for cp in issue_inputs(0, 0):
    cp.wait()
  issue_gathers(0)

  def step(s, carry):
    for b in range(2):
      c = s * 2 + b
      nxt = issue_inputs(c + 1, 1 - b) if b == 0 else None
      if b == 1:
        @pl.when(s < NCH // 2 - 1)
        def _():
          for cp in issue_inputs(c + 1, 1 - b):
            cp.wait()
      wait_gathers(b)
      compute(b)
      if b == 0:
        for cp in nxt:
          cp.wait()

        @pl.when(s > 0)
        def _():
          wait_outs(1 - b)
        issue_gathers(1 - b)
      else:
        @pl.when(s < NCH // 2 - 1)
        def _():
          wait_outs(1 - b)
          issue_gathers(1 - b)
      issue_outs(c, b)
    return carry

  lax.fori_loop(0, NCH // 2, step, 0)
  wait_outs(0)
  wait_outs(1)


@jax.jit
def _embed(poi, cid, lon, lat, id_table, ct):
  mesh = plsc.VectorSubcoreMesh(core_axis_name="c", subcore_axis_name="s")
  call = functools.partial(
      pl.kernel,
      out_type=jax.ShapeDtypeStruct((BATCH, SEQ, EMBED_DIM), jnp.float32),
      mesh=mesh,
      scratch_types=(
          [pltpu.VMEM((CATE_NUM * EMBED_DIM,), jnp.float32)]
          + [pltpu.VMEM((CHE,), jnp.int32)] * 4
          + [pltpu.VMEM((CHE,), jnp.float32)] * 4
          + [pltpu.VMEM((CHE, EMBED_DIM), jnp.float32)] * 2
          + [pltpu.SemaphoreType.DMA] * 6
      ),
      compiler_params=pltpu.CompilerParams(use_tc_tiling_on_sc=False),
  )(_body)
  return call(poi, cid, lon, lat, id_table, ct)


def kernel(poi_ids, cate_ids, lons, lats, id_table, cate_table):
  return _embed(poi_ids.astype(jnp.int32).reshape(N),
                cate_ids.astype(jnp.int32).reshape(N),
                lons.reshape(N), lats.reshape(N),
                id_table, cate_table.reshape(-1))


# speedup vs baseline: 1.3632x; 1.3632x over previous
import functools
import math

import jax
import jax.numpy as jnp
from jax import lax
from jax.experimental import pallas as pl
from jax.experimental.pallas import tpu as pltpu
from jax.experimental.pallas import tpu_sc as plsc

POI_NUM = 1000000
CATE_NUM = 1000
EMBED_DIM = 64
FREQ_DIM = 16
BATCH = 16384
SEQ = 50
N = BATCH * SEQ
NW = 32
EPT = N // NW
CHE = 400
CHR = CHE // SEQ
NCH = EPT // CHE
NG = CHE // 16

SCALE = 500.0 * math.pi
LON_MIN, LON_DEN = -180.0, 360.0
LAT_MIN, LAT_DEN = -90.0, 180.0
INV2PI = 1.0 / (2.0 * math.pi)
TWOPI = 2.0 * math.pi

S3, S5, S7 = (-0.16620338965808842, 0.00806673837856976,
              -0.0001515486170838437)
C2, C4, C6, C8 = (-0.4998795971710147, 0.04153852455139196,
                  -0.0013464001942717439, 1.9172344065239433e-05)


def _f32(x):
  return jnp.float32(x)


def _sincos(r):
  x2 = r * r
  s = _f32(S7)
  s = s * x2 + _f32(S5)
  s = s * x2 + _f32(S3)
  s = s * x2 + _f32(1.0)
  s = s * r
  c = _f32(C8)
  c = c * x2 + _f32(C6)
  c = c * x2 + _f32(C4)
  c = c * x2 + _f32(C2)
  c = c * x2 + _f32(1.0)
  return s, c


def _body(poi_h, cid_h, lon_h, lat_h, idt_h, ct_h, out_h,
          ctv, idxv0, idxv1, cidv0, cidv1, lonv0, lonv1, latv0, latv1,
          rows0, rows1, sin0, sin1, sg0, sg1, so0, so1):
  wid = lax.axis_index("s") * 2 + lax.axis_index("c")
  ebase = wid * EPT
  rbase = wid * (EPT // SEQ)
  idxv = (idxv0, idxv1)
  cidv = (cidv0, cidv1)
  lonv = (lonv0, lonv1)
  latv = (latv0, latv1)
  rows = (rows0, rows1)
  sin_ = (sin0, sin1)
  sg = (sg0, sg1)
  so = (so0, so1)

  pltpu.sync_copy(ct_h, ctv)

  iota = lax.iota(jnp.int32, 16)
  freqs = jnp.exp(iota.astype(jnp.float32) * _f32(-(math.log(10000.0) / FREQ_DIM)))
  lon_f1 = freqs * _f32(SCALE / LON_DEN)
  lon_g1 = freqs * _f32(-LON_MIN / LON_DEN * SCALE)
  lat_f1 = freqs * _f32(SCALE / LAT_DEN)
  lat_g1 = freqs * _f32(-LAT_MIN / LAT_DEN * SCALE)
  lon_f2 = lon_f1 * _f32(INV2PI)
  lon_g2 = lon_g1 * _f32(INV2PI) + _f32(0.5)
  lat_f2 = lat_f1 * _f32(INV2PI)
  lat_g2 = lat_g1 * _f32(INV2PI) + _f32(0.5)

  GSPLIT = ((0, 128), (128, 128), (256, 128), (384, 16))

  def issue_inputs(c, b):
    e0 = ebase + c * CHE
    return [pltpu.async_copy(src.at[pl.ds(e0, CHE)], dst, sin_[b])
            for src, dst in ((poi_h, idxv[b]), (cid_h, cidv[b]),
                             (lon_h, lonv[b]), (lat_h, latv[b]))]

  def issue_gathers(b):
    for o, n in GSPLIT:
      pltpu.async_copy(idt_h.at[idxv[b].at[pl.ds(o, n)]],
                       rows[b].at[pl.ds(o, n)], sg[b])

  def wait_gathers(b):
    for o, n in GSPLIT:
      pltpu.make_async_copy(idt_h.at[idxv[b].at[pl.ds(o, n)]],
                            rows[b].at[pl.ds(o, n)], sg[b]).wait()

  def issue_outs(c, b):
    r0 = rbase + c * CHR
    for j in range(CHR):
      pltpu.async_copy(rows[b].at[pl.ds(j * SEQ, SEQ)], out_h.at[r0 + j],
                       so[b])

  def wait_outs(b):
    for j in range(CHR):
      pltpu.make_async_copy(rows[b].at[pl.ds(j * SEQ, SEQ)],
                            out_h.at[rbase + j], so[b]).wait()

  def trig(v, f1, g1, f2, g2):
    a = v * f1 + g1
    t = v * f2 + g2
    nf = t.astype(jnp.int32).astype(jnp.float32)
    return _sincos(a - nf * _f32(TWOPI))

  def compute(b):
    def grp(t, _, b=b):
      i0 = t * 16
      cidvec = cidv[b][pl.ds(i0, 16)]
      lonvec = lonv[b][pl.ds(i0, 16)]
      latvec = latv[b][pl.ds(i0, 16)]
      for e in range(16):
        lv = jnp.full((16,), lonvec[e], jnp.float32)
        tv = jnp.full((16,), latvec[e], jnp.float32)
        sl, cl = trig(lv, lon_f1, lon_g1, lon_f2, lon_g2)
        st, ct = trig(tv, lat_f1, lat_g1, lat_f2, lat_g2)
        cb = cidvec[e] * jnp.int32(EMBED_DIM)
        i = i0 + e
        for k, tg in enumerate((sl, cl, st, ct)):
          cate_k = ctv[pl.ds(cb + k * 16, 16)]
          cur = rows[b][i, pl.ds(k * 16, 16)]
          rows[b][i, pl.ds(k * 16, 16)] = cur + cate_k + tg
      return 0

    lax.fori_loop(0, NG, grp, 0)

  for cp in issue_inputs(0, 0):
    cp.wait()
  issue_gathers(0)

  def step(s, carry):
    for b in range(2):
      c = s * 2 + b
      if b == 0:
        for cp in issue_inputs(c + 1, 1 - b):
          cp.wait()

        @pl.when(s > 0)
        def _():
          wait_outs(1 - b)
        issue_gathers(1 - b)
      else:
        @pl.when(s < NCH // 2 - 1)
        def _():
          for cp in issue_inputs(c + 1, 1 - b):
            cp.wait()
          wait_outs(1 - b)
          issue_gathers(1 - b)
      wait_gathers(b)
      compute(b)
      issue_outs(c, b)
    return carry

  lax.fori_loop(0, NCH // 2, step, 0)
  wait_outs(0)
  wait_outs(1)


@jax.jit
def _embed(poi, cid, lon, lat, id_table, ct):
  mesh = plsc.VectorSubcoreMesh(core_axis_name="c", subcore_axis_name="s")
  call = functools.partial(
      pl.kernel,
      out_type=jax.ShapeDtypeStruct((BATCH, SEQ, EMBED_DIM), jnp.float32),
      mesh=mesh,
      scratch_types=(
          [pltpu.VMEM((CATE_NUM * EMBED_DIM,), jnp.float32)]
          + [pltpu.VMEM((CHE,), jnp.int32)] * 4
          + [pltpu.VMEM((CHE,), jnp.float32)] * 4
          + [pltpu.VMEM((CHE, EMBED_DIM), jnp.float32)] * 2
          + [pltpu.SemaphoreType.DMA] * 6
      ),
      compiler_params=pltpu.CompilerParams(use_tc_tiling_on_sc=False),
  )(_body)
  return call(poi, cid, lon, lat, id_table, ct)


def kernel(poi_ids, cate_ids, lons, lats, id_table, cate_table):
  return _embed(poi_ids.astype(jnp.int32).reshape(N),
                cate_ids.astype(jnp.int32).reshape(N),
                lons.reshape(N), lats.reshape(N),
                id_table, cate_table.reshape(-1))
